# baseline (speedup 1.0000x reference)
import jax
import jax.numpy as jnp
from jax.experimental import pallas as pl


def _graph_ln(x, batch_ids, w, b, G):
    cnt = jax.ops.segment_sum(jnp.ones((x.shape[0],), x.dtype), batch_ids, num_segments=G) * x.shape[1]
    cnt = jnp.maximum(cnt, 1.0)
    mean = jax.ops.segment_sum(x.sum(-1), batch_ids, num_segments=G) / cnt
    xc = x - mean[batch_ids][:, None]
    var = jax.ops.segment_sum((xc * xc).sum(-1), batch_ids, num_segments=G) / cnt
    inv = jax.lax.rsqrt(var + 1e-5)
    return xc * inv[batch_ids][:, None] * w + b


def _ffn_body(h2_ref, w1_ref, b1_ref, w2_ref, b2_ref, o_ref):
    h = jnp.maximum(jnp.dot(h2_ref[...], w1_ref[...], preferred_element_type=jnp.float32) + b1_ref[...], 0.0)
    o_ref[...] = jnp.dot(h, w2_ref[...], preferred_element_type=jnp.float32) + b2_ref[...]


def kernel(x, edge_attr, edge_index, batch_ids, Aw, Ab, Bw, Bb, Cw, Cb, Dw, Db, Ew, Eb, bng, bnb, Vw1, Vb1, Vw2, Vb2, Fw1, Fb1, Fw2, Fb2, n1w, n1b, n2w, n2b):
    N, H = x.shape
    G = 64
    src = edge_index[0]
    dst = edge_index[1]
    Ax = x @ Aw.T + Ab
    Bx = x @ Bw.T + Bb
    Dx = x @ Dw.T + Db
    Ex = x @ Ew.T + Eb
    Ce = edge_attr @ Cw.T + Cb
    e_ij = Dx[dst] + Ex[src] + Ce
    sig = jax.nn.sigmoid(e_ij)
    num = jax.ops.segment_sum(sig * Bx[src], dst, num_segments=N)
    den = jax.ops.segment_sum(sig, dst, num_segments=N)
    h = Ax + num / (den + 1e-6)
    mu = h.mean(axis=0)
    v = h.var(axis=0)
    h = (h - mu) * jax.lax.rsqrt(v + 1e-5) * bng + bnb
    h = jax.nn.relu(h)
    x1 = _graph_ln(0.5 * x + 0.5 * h, batch_ids, n1w, n1b, G)
    ncnt = jnp.maximum(jax.ops.segment_sum(jnp.ones((N,), jnp.float32), batch_ids, num_segments=G), 1.0)
    pooled = jax.ops.segment_sum(x1, batch_ids, num_segments=G) / ncnt[:, None]
    vn = jax.nn.relu(pooled @ Vw1.T + Vb1) @ Vw2.T + Vb2
    h2 = x1 + vn[batch_ids]
    bm = 1000
    h3 = pl.pallas_call(
        _ffn_body,
        grid=(N // bm,),
        in_specs=[
            pl.BlockSpec((bm, H), lambda i: (i, 0)),
            pl.BlockSpec((H, 2 * H), lambda i: (0, 0)),
            pl.BlockSpec((1, 2 * H), lambda i: (0, 0)),
            pl.BlockSpec((2 * H, H), lambda i: (0, 0)),
            pl.BlockSpec((1, H), lambda i: (0, 0)),
        ],
        out_specs=pl.BlockSpec((bm, H), lambda i: (i, 0)),
        out_shape=jax.ShapeDtypeStruct((N, H), jnp.float32),
    )(h2, Fw1.T, Fb1[None, :], Fw2.T, Fb2[None, :])
    out = _graph_ln(0.5 * x1 + 0.5 * h3, batch_ids, n2w, n2b, G)
    return out


# speedup vs baseline: 2.5872x; 2.5872x over previous
import functools

import jax
import jax.numpy as jnp
from jax import lax
from jax.experimental import pallas as pl
from jax.experimental.pallas import tpu as pltpu
from jax.experimental.pallas import tpu_sc as plsc

N = 10000
E = 160000
H = 256
G = 64
NCHUNK = 4
CW = H // NCHUNK

NSC = 2
NTILE = 16
KB = 80
EPT = E // NTILE
NBATCH = EPT // KB
STRIPE = 624
ZROWS = 16

_f32 = jnp.float32


def _mm(a, b):
    return jnp.dot(a, b, preferred_element_type=_f32)


def _proj_body(x_ref, awt, ab, bwt, bb, dwt, db, ewt, eb, ax_o, *outs):
    x = x_ref[...]
    ax_o[...] = _mm(x, awt[...]) + ab[...]
    bx = _mm(x, bwt[...]) + bb[...]
    dx = _mm(x, dwt[...]) + db[...]
    ex = _mm(x, ewt[...]) + eb[...]
    for c in range(NCHUNK):
        sl = slice(c * CW, (c + 1) * CW)
        outs[c][...] = jnp.concatenate([ex[:, sl], bx[:, sl]], axis=1)
    outs[NCHUNK][...] = dx[:, :2 * CW]
    outs[NCHUNK + 1][...] = dx[:, 2 * CW:]


def _node_proj(x, AwT, Ab, BwT, Bb, DwT, Db, EwT, Eb):
    bm = 1000
    nb = N // bm
    full = pl.BlockSpec((bm, H), lambda i: (i, 0))
    w = pl.BlockSpec((H, H), lambda i: (0, 0))
    bsp = pl.BlockSpec((1, H), lambda i: (0, 0))
    ck = pl.BlockSpec((bm, 2 * CW), lambda i: (i, 0))
    return pl.pallas_call(
        _proj_body,
        grid=(nb,),
        in_specs=[full, w, bsp, w, bsp, w, bsp, w, bsp],
        out_specs=[full] + [ck] * (NCHUNK + 2),
        out_shape=[jax.ShapeDtypeStruct((N, H), _f32)]
        + [jax.ShapeDtypeStruct((N, 2 * CW), _f32)] * (NCHUNK + 2),
    )(x, AwT, Ab[None, :], BwT, Bb[None, :], DwT, Db[None, :], EwT, Eb[None, :])


def _ce_body(ea_ref, cwt, cb, o0, o1):
    ce = _mm(ea_ref[...], cwt[...]) + cb[...]
    o0[...] = ce[:, :2 * CW]
    o1[...] = ce[:, 2 * CW:]


def _edge_proj(edge_attr, CwT, Cb):
    be = 1000
    return pl.pallas_call(
        _ce_body,
        grid=(E // be,),
        in_specs=[
            pl.BlockSpec((be, H), lambda i: (i, 0)),
            pl.BlockSpec((H, H), lambda i: (0, 0)),
            pl.BlockSpec((1, H), lambda i: (0, 0)),
        ],
        out_specs=[pl.BlockSpec((be, 2 * CW), lambda i: (i, 0))] * 2,
        out_shape=[jax.ShapeDtypeStruct((E, 2 * CW), _f32)] * 2,
    )(edge_attr, CwT, Cb[None, :])


def _sc_edge_body(src, dst, eb0, eb1, eb2, eb3, dd0, dd1, ce0, ce1, nd_out,
                  acc, zbuf, idxs, idxd, bufEB, bufD, bufC, stg, sem):
    core = lax.axis_index("c")
    sub = lax.axis_index("s")
    r0 = sub * STRIPE
    ebase = sub * EPT

    def zrow(r, _):
        for cc in range(8):
            zbuf[r, pl.ds(cc * 16, 16)] = jnp.zeros((16,), _f32)
        return 0
    lax.fori_loop(0, ZROWS, zrow, 0)

    last = sub == NTILE - 1
    nzero = jnp.where(last, 40, 39)

    def chunk_pass(teb, tdd, tce, half, cg):
        hof = half * CW

        def zcp(j, _):
            pltpu.sync_copy(zbuf, acc.at[pl.ds(r0 + j * ZROWS, ZROWS)])
            return 0
        lax.fori_loop(0, nzero, zcp, 0)
        plsc.subcore_barrier()

        def batch(b, _):
            off = ebase + b * KB
            pltpu.sync_copy(src.at[pl.ds(off, KB)], idxs)
            pltpu.sync_copy(dst.at[pl.ds(off, KB)], idxd)
            cpEB = pltpu.async_copy(teb.at[idxs], bufEB, sem)
            cpD = pltpu.async_copy(tdd.at[idxd], bufD, sem)
            cpC = pltpu.async_copy(tce.at[pl.ds(off, KB)], bufC, sem)
            cpEB.wait()
            cpD.wait()
            cpC.wait()

            def row(r, _):
                for cc in range(CW // 16):
                    sl = pl.ds(cc * 16, 16)
                    slh = pl.ds(hof + cc * 16, 16)
                    slb = pl.ds(CW + cc * 16, 16)
                    t = bufD[r, slh] + bufEB[r, sl] + bufC[r, slh]
                    s = 1.0 / (1.0 + jnp.exp(-t))
                    stg[r, slb] = s
                    stg[r, sl] = s * bufEB[r, slb]
                return 0
            lax.fori_loop(0, KB, row, 0)
            pltpu.sync_copy(stg, acc.at[idxd], add=True)
            return 0
        lax.fori_loop(0, NBATCH, batch, 0)
        plsc.subcore_barrier()

        @pl.when(jnp.logical_not(last))
        def _():
            pltpu.sync_copy(acc.at[pl.ds(r0, STRIPE)],
                            nd_out.at[cg, pl.ds(r0, STRIPE)])

        @pl.when(last)
        def _():
            pltpu.sync_copy(acc.at[pl.ds(r0, STRIPE + 16)],
                            nd_out.at[cg, pl.ds(r0, STRIPE + 16)])
        plsc.subcore_barrier()

    @pl.when(core == 0)
    def _():
        chunk_pass(eb0, dd0, ce0, 0, 0)
        chunk_pass(eb1, dd0, ce0, 1, 1)

    @pl.when(core == 1)
    def _():
        chunk_pass(eb2, dd1, ce1, 0, 2)
        chunk_pass(eb3, dd1, ce1, 1, 3)


def _sc_edge(src, dst, ebs, dds, ces):
    mesh = plsc.VectorSubcoreMesh(
        core_axis_name="c", subcore_axis_name="s",
        num_cores=NSC, num_subcores=NTILE)
    fn = pl.kernel(
        _sc_edge_body,
        out_type=jax.ShapeDtypeStruct((NCHUNK, N, 2 * CW), _f32),
        mesh=mesh,
        scratch_types=[
            pltpu.VMEM_SHARED((N, 2 * CW), _f32),
            pltpu.VMEM((ZROWS, 2 * CW), _f32),
            pltpu.VMEM((KB,), jnp.int32),
            pltpu.VMEM((KB,), jnp.int32),
            pltpu.VMEM((KB, 2 * CW), _f32),
            pltpu.VMEM((KB, 2 * CW), _f32),
            pltpu.VMEM((KB, 2 * CW), _f32),
            pltpu.VMEM((KB, 2 * CW), _f32),
            pltpu.SemaphoreType.DMA,
        ],
    )
    return fn(src, dst, *ebs, *dds, *ces)


def _h_body(ax_ref, nd_ref, h_o, st_o, accum):
    i = pl.program_id(0)
    nd = nd_ref[...]
    num = jnp.concatenate([nd[c, :, :CW] for c in range(NCHUNK)], axis=1)
    den = jnp.concatenate([nd[c, :, CW:] for c in range(NCHUNK)], axis=1)
    h = ax_ref[...] + num / (den + 1e-6)
    h_o[...] = h

    @pl.when(i == 0)
    def _():
        accum[...] = jnp.zeros_like(accum)
    accum[0, :] += jnp.sum(h, axis=0)
    accum[1, :] += jnp.sum(h * h, axis=0)

    @pl.when(i == pl.num_programs(0) - 1)
    def _():
        st_o[...] = accum[...]


def _h_bn(Ax, nd):
    bm = 1000
    return pl.pallas_call(
        _h_body,
        grid=(N // bm,),
        in_specs=[
            pl.BlockSpec((bm, H), lambda i: (i, 0)),
            pl.BlockSpec((NCHUNK, bm, 2 * CW), lambda i: (0, i, 0)),
        ],
        out_specs=[
            pl.BlockSpec((bm, H), lambda i: (i, 0)),
            pl.BlockSpec((2, H), lambda i: (0, 0)),
        ],
        out_shape=[
            jax.ShapeDtypeStruct((N, H), _f32),
            jax.ShapeDtypeStruct((2, H), _f32),
        ],
        scratch_shapes=[pltpu.VMEM((2, H), _f32)],
    )(Ax, nd)


def _onehot(ids, bm):
    return (ids == lax.broadcasted_iota(jnp.int32, (bm, G), 1)).astype(_f32)


def _gpart(oh, y, bm):
    rs = jnp.sum(y, axis=1, keepdims=True)
    rss = jnp.sum(y * y, axis=1, keepdims=True)
    ones = jnp.ones((bm, 1), _f32)
    zeros = jnp.zeros((bm, 5), _f32)
    part = jnp.concatenate([rs, rss, ones, zeros], axis=1)
    return lax.dot_general(oh, part, (((0,), (0,)), ((), ())),
                           preferred_element_type=_f32)


def _y_body(h_ref, st_ref, x_ref, ids_ref, bng, bnb, y_o, gs_o, accum):
    i = pl.program_id(0)
    bm = h_ref.shape[0]
    st = st_ref[...]
    mu = st[0:1, :] / N
    var = st[1:2, :] / N - mu * mu
    h = (h_ref[...] - mu) * lax.rsqrt(var + 1e-5) * bng[...] + bnb[...]
    h = jnp.maximum(h, 0.0)
    y = 0.5 * x_ref[...] + 0.5 * h
    y_o[...] = y
    oh = _onehot(ids_ref[...], bm)

    @pl.when(i == 0)
    def _():
        accum[...] = jnp.zeros_like(accum)
    accum[...] += _gpart(oh, y, bm)

    @pl.when(i == pl.num_programs(0) - 1)
    def _():
        gs_o[...] = accum[...]


def _y_gstats(h, stats, x, ids2d, bng, bnb):
    bm = 1000
    return pl.pallas_call(
        _y_body,
        grid=(N // bm,),
        in_specs=[
            pl.BlockSpec((bm, H), lambda i: (i, 0)),
            pl.BlockSpec((2, H), lambda i: (0, 0)),
            pl.BlockSpec((bm, H), lambda i: (i, 0)),
            pl.BlockSpec((bm, 1), lambda i: (i, 0)),
            pl.BlockSpec((1, H), lambda i: (0, 0)),
            pl.BlockSpec((1, H), lambda i: (0, 0)),
        ],
        out_specs=[
            pl.BlockSpec((bm, H), lambda i: (i, 0)),
            pl.BlockSpec((G, 8), lambda i: (0, 0)),
        ],
        out_shape=[
            jax.ShapeDtypeStruct((N, H), _f32),
            jax.ShapeDtypeStruct((G, 8), _f32),
        ],
        scratch_shapes=[pltpu.VMEM((G, 8), _f32)],
    )(h, stats, x, ids2d, bng[None, :], bnb[None, :])


def _ln_node_stats(gs, oh):
    s1 = gs[:, 0:1]
    s2 = gs[:, 1:2]
    n = gs[:, 2:3]
    cnt = jnp.maximum(n * H, 1.0)
    mean = s1 / cnt
    var = s2 / cnt - mean * mean
    inv = lax.rsqrt(var + 1e-5)
    sts = jnp.concatenate([mean, inv, jnp.zeros((G, 6), _f32)], axis=1)
    node = _mm(oh, sts)
    return node[:, 0:1], node[:, 1:2]


def _x1_body(y_ref, gs_ref, ids_ref, n1w, n1b, x1_o, pool_o, accum):
    i = pl.program_id(0)
    bm = y_ref.shape[0]
    oh = _onehot(ids_ref[...], bm)
    mean_n, inv_n = _ln_node_stats(gs_ref[...], oh)
    x1 = (y_ref[...] - mean_n) * inv_n * n1w[...] + n1b[...]
    x1_o[...] = x1

    @pl.when(i == 0)
    def _():
        accum[...] = jnp.zeros_like(accum)
    accum[...] += lax.dot_general(oh, x1, (((0,), (0,)), ((), ())),
                                  preferred_element_type=_f32)

    @pl.when(i == pl.num_programs(0) - 1)
    def _():
        pool_o[...] = accum[...]


def _x1_pool(y, gs1, ids2d, n1w, n1b):
    bm = 1000
    return pl.pallas_call(
        _x1_body,
        grid=(N // bm,),
        in_specs=[
            pl.BlockSpec((bm, H), lambda i: (i, 0)),
            pl.BlockSpec((G, 8), lambda i: (0, 0)),
            pl.BlockSpec((bm, 1), lambda i: (i, 0)),
            pl.BlockSpec((1, H), lambda i: (0, 0)),
            pl.BlockSpec((1, H), lambda i: (0, 0)),
        ],
        out_specs=[
            pl.BlockSpec((bm, H), lambda i: (i, 0)),
            pl.BlockSpec((G, H), lambda i: (0, 0)),
        ],
        out_shape=[
            jax.ShapeDtypeStruct((N, H), _f32),
            jax.ShapeDtypeStruct((G, H), _f32),
        ],
        scratch_shapes=[pltpu.VMEM((G, H), _f32)],
    )(y, gs1, ids2d, n1w[None, :], n1b[None, :])


def _ffn_body(x1_ref, pool_ref, gs_ref, ids_ref, vw1t, vb1, vw2t, vb2,
              fw1t, fb1, fw2t, fb2, z_o, gs2_o, vn_s, accum):
    i = pl.program_id(0)
    bm = x1_ref.shape[0]

    @pl.when(i == 0)
    def _():
        n = jnp.maximum(gs_ref[...][:, 2:3], 1.0)
        pooled = pool_ref[...] / n
        hv = jnp.maximum(_mm(pooled, vw1t[...]) + vb1[...], 0.0)
        vn_s[...] = _mm(hv, vw2t[...]) + vb2[...]
        accum[...] = jnp.zeros_like(accum)

    oh = _onehot(ids_ref[...], bm)
    x1 = x1_ref[...]
    h2 = x1 + _mm(oh, vn_s[...])
    u = jnp.maximum(_mm(h2, fw1t[...]) + fb1[...], 0.0)
    u = _mm(u, fw2t[...]) + fb2[...]
    z = 0.5 * x1 + 0.5 * u
    z_o[...] = z
    accum[...] += _gpart(oh, z, bm)

    @pl.when(i == pl.num_programs(0) - 1)
    def _():
        gs2_o[...] = accum[...]


def _ffn(x1, pool, gs1, ids2d, Vw1T, Vb1, Vw2T, Vb2, Fw1T, Fb1, Fw2T, Fb2):
    bm = 1000
    cst = lambda shape: pl.BlockSpec(shape, lambda i: tuple(0 for _ in shape))
    return pl.pallas_call(
        _ffn_body,
        grid=(N // bm,),
        in_specs=[
            pl.BlockSpec((bm, H), lambda i: (i, 0)),
            cst((G, H)),
            cst((G, 8)),
            pl.BlockSpec((bm, 1), lambda i: (i, 0)),
            cst((H, H)), cst((1, H)), cst((H, H)), cst((1, H)),
            cst((H, 2 * H)), cst((1, 2 * H)), cst((2 * H, H)), cst((1, H)),
        ],
        out_specs=[
            pl.BlockSpec((bm, H), lambda i: (i, 0)),
            cst((G, 8)),
        ],
        out_shape=[
            jax.ShapeDtypeStruct((N, H), _f32),
            jax.ShapeDtypeStruct((G, 8), _f32),
        ],
        scratch_shapes=[pltpu.VMEM((G, H), _f32), pltpu.VMEM((G, 8), _f32)],
    )(x1, pool, gs1, ids2d, Vw1T, Vb1[None, :], Vw2T, Vb2[None, :],
      Fw1T, Fb1[None, :], Fw2T, Fb2[None, :])


def _fin_body(z_ref, gs_ref, ids_ref, n2w, n2b, o_ref):
    bm = z_ref.shape[0]
    oh = _onehot(ids_ref[...], bm)
    mean_n, inv_n = _ln_node_stats(gs_ref[...], oh)
    o_ref[...] = (z_ref[...] - mean_n) * inv_n * n2w[...] + n2b[...]


def _final(z, gs2, ids2d, n2w, n2b):
    bm = 1000
    return pl.pallas_call(
        _fin_body,
        grid=(N // bm,),
        in_specs=[
            pl.BlockSpec((bm, H), lambda i: (i, 0)),
            pl.BlockSpec((G, 8), lambda i: (0, 0)),
            pl.BlockSpec((bm, 1), lambda i: (i, 0)),
            pl.BlockSpec((1, H), lambda i: (0, 0)),
            pl.BlockSpec((1, H), lambda i: (0, 0)),
        ],
        out_specs=pl.BlockSpec((bm, H), lambda i: (i, 0)),
        out_shape=jax.ShapeDtypeStruct((N, H), _f32),
    )(z, gs2, ids2d, n2w[None, :], n2b[None, :])


def kernel(x, edge_attr, edge_index, batch_ids, Aw, Ab, Bw, Bb, Cw, Cb,
           Dw, Db, Ew, Eb, bng, bnb, Vw1, Vb1, Vw2, Vb2, Fw1, Fb1, Fw2, Fb2,
           n1w, n1b, n2w, n2b):
    src = edge_index[0]
    dst = edge_index[1]
    ids2d = batch_ids.reshape(N, 1)

    proj = _node_proj(x, Aw.T, Ab, Bw.T, Bb, Dw.T, Db, Ew.T, Eb)
    Ax = proj[0]
    ebs = proj[1:1 + NCHUNK]
    dds = proj[1 + NCHUNK:3 + NCHUNK]
    ces = _edge_proj(edge_attr, Cw.T, Cb)

    nd = _sc_edge(src, dst, ebs, dds, ces)

    h, stats = _h_bn(Ax, nd)
    y, gs1 = _y_gstats(h, stats, x, ids2d, bng, bnb)
    x1, pool = _x1_pool(y, gs1, ids2d, n1w, n1b)
    z, gs2 = _ffn(x1, pool, gs1, ids2d, Vw1.T, Vb1, Vw2.T, Vb2,
                  Fw1.T, Fb1, Fw2.T, Fb2)
    return _final(z, gs2, ids2d, n2w, n2b)


# speedup vs baseline: 3.9270x; 1.5179x over previous
import functools

import jax
import jax.numpy as jnp
from jax import lax
from jax.experimental import pallas as pl
from jax.experimental.pallas import tpu as pltpu
from jax.experimental.pallas import tpu_sc as plsc

N = 10000
E = 160000
H = 256
G = 64
NCHUNK = 4
CW = H // NCHUNK

NSC = 2
NTILE = 16
KB = 32
EPTA = 9984
EPTB = E - 15 * EPTA
NPAIRA = EPTA // KB // 2
NPAIRB = EPTB // KB // 2
STRIPE = 624
ZROWS = 16

_f32 = jnp.float32


def _mm(a, b):
    return jnp.dot(a, b, preferred_element_type=_f32)


def _proj_body(x_ref, awt, ab, bwt, bb, dwt, db, ewt, eb, ax_o, *outs):
    x = x_ref[...]
    ax_o[...] = _mm(x, awt[...]) + ab[...]
    bx = _mm(x, bwt[...]) + bb[...]
    dx = _mm(x, dwt[...]) + db[...]
    ex = _mm(x, ewt[...]) + eb[...]
    for c in range(NCHUNK):
        sl = slice(c * CW, (c + 1) * CW)
        outs[c][...] = jnp.concatenate([ex[:, sl], bx[:, sl]], axis=1)
    outs[NCHUNK][...] = dx[:, :2 * CW]
    outs[NCHUNK + 1][...] = dx[:, 2 * CW:]


def _node_proj(x, AwT, Ab, BwT, Bb, DwT, Db, EwT, Eb):
    bm = 1000
    nb = N // bm
    full = pl.BlockSpec((bm, H), lambda i: (i, 0))
    w = pl.BlockSpec((H, H), lambda i: (0, 0))
    bsp = pl.BlockSpec((1, H), lambda i: (0, 0))
    ck = pl.BlockSpec((bm, 2 * CW), lambda i: (i, 0))
    return pl.pallas_call(
        _proj_body,
        grid=(nb,),
        in_specs=[full, w, bsp, w, bsp, w, bsp, w, bsp],
        out_specs=[full] + [ck] * (NCHUNK + 2),
        out_shape=[jax.ShapeDtypeStruct((N, H), _f32)]
        + [jax.ShapeDtypeStruct((N, 2 * CW), _f32)] * (NCHUNK + 2),
    )(x, AwT, Ab[None, :], BwT, Bb[None, :], DwT, Db[None, :], EwT, Eb[None, :])


def _ce_body(ea_ref, cwt, cb, o0, o1):
    ce = _mm(ea_ref[...], cwt[...]) + cb[...]
    o0[...] = ce[:, :2 * CW]
    o1[...] = ce[:, 2 * CW:]


def _edge_proj(edge_attr, CwT, Cb):
    be = 1000
    return pl.pallas_call(
        _ce_body,
        grid=(E // be,),
        in_specs=[
            pl.BlockSpec((be, H), lambda i: (i, 0)),
            pl.BlockSpec((H, H), lambda i: (0, 0)),
            pl.BlockSpec((1, H), lambda i: (0, 0)),
        ],
        out_specs=[pl.BlockSpec((be, 2 * CW), lambda i: (i, 0))] * 2,
        out_shape=[jax.ShapeDtypeStruct((E, 2 * CW), _f32)] * 2,
    )(edge_attr, CwT, Cb[None, :])


def _sc_edge_body(src, dst, eb0, eb1, eb2, eb3, dd0, dd1, ce0, ce1, nd_out,
                  acc, zbuf,
                  ixs0, ixd0, ixs1, ixd1, sx0, sx1,
                  beb0, bdd0, bce0, beb1, bdd1, bce1, stg0, stg1,
                  semG0, semG1, semI0, semI1, semS0, semS1):
    core = lax.axis_index("c")
    sub = lax.axis_index("s")
    r0 = sub * STRIPE
    ebase = sub * EPTA

    slot0 = (ixs0, ixd0, sx0, beb0, bdd0, bce0, stg0, semG0, semI0, semS0)
    slot1 = (ixs1, ixd1, sx1, beb1, bdd1, bce1, stg1, semG1, semI1, semS1)

    def zrow(r, _):
        for cc in range(8):
            zbuf[r, pl.ds(cc * 16, 16)] = jnp.zeros((16,), _f32)
        return 0
    lax.fori_loop(0, ZROWS, zrow, 0)

    last = sub == NTILE - 1
    nzero = jnp.where(last, 40, 39)

    def chunk_pass(teb, tdd, tce, half, cg):
        hof = half * CW

        def i_issue(slot, off):
            ixs, ixd = slot[0], slot[1]
            semI = slot[8]
            pltpu.async_copy(src.at[pl.ds(off, KB)], ixs, semI)
            pltpu.async_copy(dst.at[pl.ds(off, KB)], ixd, semI)

        def i_wait(slot):
            ixs, ixd = slot[0], slot[1]
            semI = slot[8]
            pltpu.make_async_copy(src.at[pl.ds(0, KB)], ixs, semI).wait()
            pltpu.make_async_copy(dst.at[pl.ds(0, KB)], ixd, semI).wait()

        def g_issue(slot, off):
            ixs, ixd, beb, bdd, bce = slot[0], slot[1], slot[3], slot[4], slot[5]
            semG = slot[7]
            pltpu.async_copy(teb.at[ixs], beb, semG)
            pltpu.async_copy(tdd.at[ixd], bdd, semG)
            pltpu.async_copy(tce.at[pl.ds(off, KB)], bce, semG)

        def g_wait(slot):
            ixs, ixd, beb, bdd, bce = slot[0], slot[1], slot[3], slot[4], slot[5]
            semG = slot[7]
            pltpu.make_async_copy(teb.at[ixs], beb, semG).wait()
            pltpu.make_async_copy(tdd.at[ixd], bdd, semG).wait()
            pltpu.make_async_copy(tce.at[pl.ds(0, KB)], bce, semG).wait()

        def s_issue(slot):
            sx, stg, semS = slot[2], slot[6], slot[9]
            pltpu.async_copy(stg, acc.at[sx], semS, add=True)

        def s_wait(slot):
            sx, stg, semS = slot[2], slot[6], slot[9]
            pltpu.make_async_copy(stg, acc.at[sx], semS).wait()

        def copy_sx(slot):
            ixd, sx = slot[1], slot[2]
            for j in range(KB // 16):
                sx[pl.ds(j * 16, 16)] = ixd[pl.ds(j * 16, 16)]

        def compute(slot):
            beb, bdd, bce, stg = slot[3], slot[4], slot[5], slot[6]

            def row(r, _):
                for cc in range(CW // 16):
                    sl = pl.ds(cc * 16, 16)
                    slh = pl.ds(hof + cc * 16, 16)
                    slb = pl.ds(CW + cc * 16, 16)
                    t = bdd[r, slh] + beb[r, sl] + bce[r, slh]
                    s = 1.0 / (1.0 + jnp.exp(-t))
                    stg[r, slb] = s
                    stg[r, sl] = s * beb[r, slb]
                return 0
            lax.fori_loop(0, KB, row, 0)

        def zcp(j, _):
            pltpu.sync_copy(zbuf, acc.at[pl.ds(r0 + j * ZROWS, ZROWS)])
            return 0
        lax.fori_loop(0, nzero, zcp, 0)
        plsc.subcore_barrier()

        npair = jnp.where(last, NPAIRB, NPAIRA)
        pltpu.sync_copy(src.at[pl.ds(ebase, KB)], ixs0)
        pltpu.sync_copy(dst.at[pl.ds(ebase, KB)], ixd0)
        g_issue(slot0, ebase)
        i_issue(slot1, ebase + KB)

        def pair(k, _):
            off = ebase + 2 * k * KB
            more = k < npair - 1
            i_wait(slot1)
            g_issue(slot1, off + KB)

            g_wait(slot0)

            @pl.when(k > 0)
            def _():
                s_wait(slot0)
            copy_sx(slot0)

            @pl.when(more)
            def _():
                i_issue(slot0, off + 2 * KB)
            compute(slot0)
            s_issue(slot0)

            @pl.when(more)
            def _():
                i_wait(slot0)
                g_issue(slot0, off + 2 * KB)

            g_wait(slot1)

            @pl.when(k > 0)
            def _():
                s_wait(slot1)
            copy_sx(slot1)

            @pl.when(more)
            def _():
                i_issue(slot1, off + 3 * KB)
            compute(slot1)
            s_issue(slot1)
            return 0
        lax.fori_loop(0, npair, pair, 0)

        s_wait(slot0)
        s_wait(slot1)
        plsc.subcore_barrier()

        @pl.when(jnp.logical_not(last))
        def _():
            pltpu.sync_copy(acc.at[pl.ds(r0, STRIPE)],
                            nd_out.at[cg, pl.ds(r0, STRIPE)])

        @pl.when(last)
        def _():
            pltpu.sync_copy(acc.at[pl.ds(r0, STRIPE + 16)],
                            nd_out.at[cg, pl.ds(r0, STRIPE + 16)])
        plsc.subcore_barrier()

    @pl.when(core == 0)
    def _():
        chunk_pass(eb0, dd0, ce0, 0, 0)
        chunk_pass(eb1, dd0, ce0, 1, 1)

    @pl.when(core == 1)
    def _():
        chunk_pass(eb2, dd1, ce1, 0, 2)
        chunk_pass(eb3, dd1, ce1, 1, 3)


def _sc_edge(src, dst, ebs, dds, ces):
    mesh = plsc.VectorSubcoreMesh(
        core_axis_name="c", subcore_axis_name="s",
        num_cores=NSC, num_subcores=NTILE)
    fn = pl.kernel(
        _sc_edge_body,
        out_type=jax.ShapeDtypeStruct((NCHUNK, N, 2 * CW), _f32),
        mesh=mesh,
        scratch_types=(
            [
                pltpu.VMEM_SHARED((N, 2 * CW), _f32),
                pltpu.VMEM((ZROWS, 2 * CW), _f32),
            ]
            + [pltpu.VMEM((KB,), jnp.int32)] * 6
            + [pltpu.VMEM((KB, 2 * CW), _f32)] * 8
            + [pltpu.SemaphoreType.DMA] * 6
        ),
    )
    return fn(src, dst, *ebs, *dds, *ces)


def _h_body(ax_ref, nd_ref, h_o, st_o, accum):
    i = pl.program_id(0)
    nd = nd_ref[...]
    num = jnp.concatenate([nd[c, :, :CW] for c in range(NCHUNK)], axis=1)
    den = jnp.concatenate([nd[c, :, CW:] for c in range(NCHUNK)], axis=1)
    h = ax_ref[...] + num / (den + 1e-6)
    h_o[...] = h

    @pl.when(i == 0)
    def _():
        accum[...] = jnp.zeros_like(accum)
    accum[0, :] += jnp.sum(h, axis=0)
    accum[1, :] += jnp.sum(h * h, axis=0)

    @pl.when(i == pl.num_programs(0) - 1)
    def _():
        st_o[...] = accum[...]


def _h_bn(Ax, nd):
    bm = 1000
    return pl.pallas_call(
        _h_body,
        grid=(N // bm,),
        in_specs=[
            pl.BlockSpec((bm, H), lambda i: (i, 0)),
            pl.BlockSpec((NCHUNK, bm, 2 * CW), lambda i: (0, i, 0)),
        ],
        out_specs=[
            pl.BlockSpec((bm, H), lambda i: (i, 0)),
            pl.BlockSpec((2, H), lambda i: (0, 0)),
        ],
        out_shape=[
            jax.ShapeDtypeStruct((N, H), _f32),
            jax.ShapeDtypeStruct((2, H), _f32),
        ],
        scratch_shapes=[pltpu.VMEM((2, H), _f32)],
    )(Ax, nd)


def _onehot(ids, bm):
    return (ids == lax.broadcasted_iota(jnp.int32, (bm, G), 1)).astype(_f32)


def _gpart(oh, y, bm):
    rs = jnp.sum(y, axis=1, keepdims=True)
    rss = jnp.sum(y * y, axis=1, keepdims=True)
    ones = jnp.ones((bm, 1), _f32)
    zeros = jnp.zeros((bm, 5), _f32)
    part = jnp.concatenate([rs, rss, ones, zeros], axis=1)
    return lax.dot_general(oh, part, (((0,), (0,)), ((), ())),
                           preferred_element_type=_f32)


def _y_body(h_ref, st_ref, x_ref, ids_ref, bng, bnb, y_o, gs_o, accum):
    i = pl.program_id(0)
    bm = h_ref.shape[0]
    st = st_ref[...]
    mu = st[0:1, :] / N
    var = st[1:2, :] / N - mu * mu
    h = (h_ref[...] - mu) * lax.rsqrt(var + 1e-5) * bng[...] + bnb[...]
    h = jnp.maximum(h, 0.0)
    y = 0.5 * x_ref[...] + 0.5 * h
    y_o[...] = y
    oh = _onehot(ids_ref[...], bm)

    @pl.when(i == 0)
    def _():
        accum[...] = jnp.zeros_like(accum)
    accum[...] += _gpart(oh, y, bm)

    @pl.when(i == pl.num_programs(0) - 1)
    def _():
        gs_o[...] = accum[...]


def _y_gstats(h, stats, x, ids2d, bng, bnb):
    bm = 1000
    return pl.pallas_call(
        _y_body,
        grid=(N // bm,),
        in_specs=[
            pl.BlockSpec((bm, H), lambda i: (i, 0)),
            pl.BlockSpec((2, H), lambda i: (0, 0)),
            pl.BlockSpec((bm, H), lambda i: (i, 0)),
            pl.BlockSpec((bm, 1), lambda i: (i, 0)),
            pl.BlockSpec((1, H), lambda i: (0, 0)),
            pl.BlockSpec((1, H), lambda i: (0, 0)),
        ],
        out_specs=[
            pl.BlockSpec((bm, H), lambda i: (i, 0)),
            pl.BlockSpec((G, 8), lambda i: (0, 0)),
        ],
        out_shape=[
            jax.ShapeDtypeStruct((N, H), _f32),
            jax.ShapeDtypeStruct((G, 8), _f32),
        ],
        scratch_shapes=[pltpu.VMEM((G, 8), _f32)],
    )(h, stats, x, ids2d, bng[None, :], bnb[None, :])


def _ln_node_stats(gs, oh):
    s1 = gs[:, 0:1]
    s2 = gs[:, 1:2]
    n = gs[:, 2:3]
    cnt = jnp.maximum(n * H, 1.0)
    mean = s1 / cnt
    var = s2 / cnt - mean * mean
    inv = lax.rsqrt(var + 1e-5)
    sts = jnp.concatenate([mean, inv, jnp.zeros((G, 6), _f32)], axis=1)
    node = _mm(oh, sts)
    return node[:, 0:1], node[:, 1:2]


def _x1_body(y_ref, gs_ref, ids_ref, n1w, n1b, x1_o, pool_o, accum):
    i = pl.program_id(0)
    bm = y_ref.shape[0]
    oh = _onehot(ids_ref[...], bm)
    mean_n, inv_n = _ln_node_stats(gs_ref[...], oh)
    x1 = (y_ref[...] - mean_n) * inv_n * n1w[...] + n1b[...]
    x1_o[...] = x1

    @pl.when(i == 0)
    def _():
        accum[...] = jnp.zeros_like(accum)
    accum[...] += lax.dot_general(oh, x1, (((0,), (0,)), ((), ())),
                                  preferred_element_type=_f32)

    @pl.when(i == pl.num_programs(0) - 1)
    def _():
        pool_o[...] = accum[...]


def _x1_pool(y, gs1, ids2d, n1w, n1b):
    bm = 1000
    return pl.pallas_call(
        _x1_body,
        grid=(N // bm,),
        in_specs=[
            pl.BlockSpec((bm, H), lambda i: (i, 0)),
            pl.BlockSpec((G, 8), lambda i: (0, 0)),
            pl.BlockSpec((bm, 1), lambda i: (i, 0)),
            pl.BlockSpec((1, H), lambda i: (0, 0)),
            pl.BlockSpec((1, H), lambda i: (0, 0)),
        ],
        out_specs=[
            pl.BlockSpec((bm, H), lambda i: (i, 0)),
            pl.BlockSpec((G, H), lambda i: (0, 0)),
        ],
        out_shape=[
            jax.ShapeDtypeStruct((N, H), _f32),
            jax.ShapeDtypeStruct((G, H), _f32),
        ],
        scratch_shapes=[pltpu.VMEM((G, H), _f32)],
    )(y, gs1, ids2d, n1w[None, :], n1b[None, :])


def _ffn_body(x1_ref, pool_ref, gs_ref, ids_ref, vw1t, vb1, vw2t, vb2,
              fw1t, fb1, fw2t, fb2, z_o, gs2_o, vn_s, accum):
    i = pl.program_id(0)
    bm = x1_ref.shape[0]

    @pl.when(i == 0)
    def _():
        n = jnp.maximum(gs_ref[...][:, 2:3], 1.0)
        pooled = pool_ref[...] / n
        hv = jnp.maximum(_mm(pooled, vw1t[...]) + vb1[...], 0.0)
        vn_s[...] = _mm(hv, vw2t[...]) + vb2[...]
        accum[...] = jnp.zeros_like(accum)

    oh = _onehot(ids_ref[...], bm)
    x1 = x1_ref[...]
    h2 = x1 + _mm(oh, vn_s[...])
    u = jnp.maximum(_mm(h2, fw1t[...]) + fb1[...], 0.0)
    u = _mm(u, fw2t[...]) + fb2[...]
    z = 0.5 * x1 + 0.5 * u
    z_o[...] = z
    accum[...] += _gpart(oh, z, bm)

    @pl.when(i == pl.num_programs(0) - 1)
    def _():
        gs2_o[...] = accum[...]


def _ffn(x1, pool, gs1, ids2d, Vw1T, Vb1, Vw2T, Vb2, Fw1T, Fb1, Fw2T, Fb2):
    bm = 1000
    cst = lambda shape: pl.BlockSpec(shape, lambda i: tuple(0 for _ in shape))
    return pl.pallas_call(
        _ffn_body,
        grid=(N // bm,),
        in_specs=[
            pl.BlockSpec((bm, H), lambda i: (i, 0)),
            cst((G, H)),
            cst((G, 8)),
            pl.BlockSpec((bm, 1), lambda i: (i, 0)),
            cst((H, H)), cst((1, H)), cst((H, H)), cst((1, H)),
            cst((H, 2 * H)), cst((1, 2 * H)), cst((2 * H, H)), cst((1, H)),
        ],
        out_specs=[
            pl.BlockSpec((bm, H), lambda i: (i, 0)),
            cst((G, 8)),
        ],
        out_shape=[
            jax.ShapeDtypeStruct((N, H), _f32),
            jax.ShapeDtypeStruct((G, 8), _f32),
        ],
        scratch_shapes=[pltpu.VMEM((G, H), _f32), pltpu.VMEM((G, 8), _f32)],
    )(x1, pool, gs1, ids2d, Vw1T, Vb1[None, :], Vw2T, Vb2[None, :],
      Fw1T, Fb1[None, :], Fw2T, Fb2[None, :])


def _fin_body(z_ref, gs_ref, ids_ref, n2w, n2b, o_ref):
    bm = z_ref.shape[0]
    oh = _onehot(ids_ref[...], bm)
    mean_n, inv_n = _ln_node_stats(gs_ref[...], oh)
    o_ref[...] = (z_ref[...] - mean_n) * inv_n * n2w[...] + n2b[...]


def _final(z, gs2, ids2d, n2w, n2b):
    bm = 1000
    return pl.pallas_call(
        _fin_body,
        grid=(N // bm,),
        in_specs=[
            pl.BlockSpec((bm, H), lambda i: (i, 0)),
            pl.BlockSpec((G, 8), lambda i: (0, 0)),
            pl.BlockSpec((bm, 1), lambda i: (i, 0)),
            pl.BlockSpec((1, H), lambda i: (0, 0)),
            pl.BlockSpec((1, H), lambda i: (0, 0)),
        ],
        out_specs=pl.BlockSpec((bm, H), lambda i: (i, 0)),
        out_shape=jax.ShapeDtypeStruct((N, H), _f32),
    )(z, gs2, ids2d, n2w[None, :], n2b[None, :])


def kernel(x, edge_attr, edge_index, batch_ids, Aw, Ab, Bw, Bb, Cw, Cb,
           Dw, Db, Ew, Eb, bng, bnb, Vw1, Vb1, Vw2, Vb2, Fw1, Fb1, Fw2, Fb2,
           n1w, n1b, n2w, n2b):
    src = edge_index[0]
    dst = edge_index[1]
    ids2d = batch_ids.reshape(N, 1)

    proj = _node_proj(x, Aw.T, Ab, Bw.T, Bb, Dw.T, Db, Ew.T, Eb)
    Ax = proj[0]
    ebs = proj[1:1 + NCHUNK]
    dds = proj[1 + NCHUNK:3 + NCHUNK]
    ces = _edge_proj(edge_attr, Cw.T, Cb)

    nd = _sc_edge(src, dst, ebs, dds, ces)

    h, stats = _h_bn(Ax, nd)
    y, gs1 = _y_gstats(h, stats, x, ids2d, bng, bnb)
    x1, pool = _x1_pool(y, gs1, ids2d, n1w, n1b)
    z, gs2 = _ffn(x1, pool, gs1, ids2d, Vw1.T, Vb1, Vw2.T, Vb2,
                  Fw1.T, Fb1, Fw2.T, Fb2)
    return _final(z, gs2, ids2d, n2w, n2b)
